# baseline (speedup 1.0000x reference)
import functools

import jax
import jax.numpy as jnp
from jax import lax
from jax.experimental import pallas as pl
from jax.experimental.pallas import tpu as pltpu
from jax.experimental.pallas import tpu_sc as plsc

N_NODES = 10000
IN_DIM = 128
D = 64
CW = 8
E = 640000

NC = 2
NS = 16
EPW = E // NS
CHUNK = 128
_G = 8
_NGROUP = 39
TAIL = EPW - _NGROUP * _G * CHUNK

_ZROWS = [640] * 15 + [400]
_ZOFF = [640 * i for i in range(16)]

_SC_PARAMS = pltpu.CompilerParams(use_tc_tiling_on_sc=False)
_MESH = dict(core_axis_name="c", subcore_axis_name="s",
             num_cores=NC, num_subcores=NS)


def _sc_agg_body(with_counts, src_a, dst_a, tab_a, src_b, dst_b, tab_b,
                 zeros64, zeros_cw, ones_hbm, *refs):
    if with_counts:
        agg_a_out, agg_b_out, cnt_a_out, cnt_b_out = refs[:4]
        refs = refs[4:]
    else:
        agg_a_out, agg_b_out = refs[:2]
        cnt_a_out = cnt_b_out = None
        refs = refs[2:]
    isx = refs[0:_G]
    idx = refs[_G:2 * _G]
    rws = refs[2 * _G:3 * _G]
    isx_t, idx_t, rws_t = refs[3 * _G:3 * _G + 3]
    refs = refs[3 * _G + 3:]
    if with_counts:
        ones_v, acc, cacc, sem_i, sem_g, sem_s = refs
    else:
        acc, sem_i, sem_g, sem_s = refs
        cacc = None

    c = lax.axis_index("c")
    s = lax.axis_index("s")
    base = s * EPW

    def run_direction(src_hbm, dst_hbm, tab_hbm, agg_out, cnt_out):
        for t in range(NS):
            @pl.when(s == t)
            def _():
                sl = pl.ds(_ZOFF[t], _ZROWS[t])
                pltpu.sync_copy(zeros64.at[pl.ds(0, _ZROWS[t])], acc.at[sl])
                if with_counts:
                    pltpu.sync_copy(zeros_cw.at[pl.ds(0, _ZROWS[t])],
                                    cacc.at[sl])
        if with_counts:
            pltpu.sync_copy(ones_hbm, ones_v)
        plsc.subcore_barrier()

        def drain_scatters(ks):
            for k in ks:
                pltpu.make_async_copy(rws[k], acc.at[idx[k]], sem_s).wait()
                if with_counts:
                    pltpu.make_async_copy(ones_v, cacc.at[idx[k]],
                                          sem_s).wait()

        def run_set(g, ks):
            @pl.when(g > 0)
            def _():
                drain_scatters(ks)
            dsi = []
            for k in ks:
                sl = pl.ds(base + (g * _G + k) * CHUNK, CHUNK)
                dsi.append(pltpu.async_copy(src_hbm.at[sl], isx[k], sem_i))
                dsi.append(pltpu.async_copy(dst_hbm.at[sl], idx[k], sem_i))
            dsg = []
            for j, k in enumerate(ks):
                dsi[2 * j].wait()
                dsi[2 * j + 1].wait()
                dsg.append(pltpu.async_copy(tab_hbm.at[isx[k]], rws[k],
                                            sem_g))
            for j, k in enumerate(ks):
                dsg[j].wait()
                pltpu.async_copy(rws[k], acc.at[idx[k]], sem_s, add=True)
                if with_counts:
                    pltpu.async_copy(ones_v, cacc.at[idx[k]], sem_s,
                                     add=True)

        half = _G // 2
        set0 = list(range(half))
        set1 = list(range(half, _G))

        def group(g, _):
            run_set(g, set0)
            run_set(g, set1)
            return 0

        lax.fori_loop(0, _NGROUP, group, 0)
        drain_scatters(set0)
        drain_scatters(set1)

        sl = pl.ds(base + _NGROUP * _G * CHUNK, TAIL)
        pltpu.sync_copy(src_hbm.at[sl], isx_t)
        pltpu.sync_copy(dst_hbm.at[sl], idx_t)
        pltpu.async_copy(tab_hbm.at[isx_t], rws_t, sem_g).wait()
        pltpu.async_copy(rws_t, acc.at[idx_t], sem_s, add=True)
        if with_counts:
            pltpu.async_copy(ones_v.at[pl.ds(0, TAIL)], cacc.at[idx_t],
                             sem_s, add=True)
            pltpu.make_async_copy(ones_v.at[pl.ds(0, TAIL)],
                                  cacc.at[idx_t], sem_s).wait()
        pltpu.make_async_copy(rws_t, acc.at[idx_t], sem_s).wait()
        plsc.subcore_barrier()

        for t in range(NS):
            @pl.when(s == t)
            def _():
                sl = pl.ds(_ZOFF[t], _ZROWS[t])
                pltpu.sync_copy(acc.at[sl], agg_out.at[sl])
                if with_counts:
                    pltpu.sync_copy(cacc.at[sl], cnt_out.at[sl])

    @pl.when(c == 0)
    def _():
        run_direction(src_a, dst_a, tab_a, agg_a_out, cnt_a_out)

    @pl.when(c == 1)
    def _():
        run_direction(src_b, dst_b, tab_b, agg_b_out, cnt_b_out)


def _make_sc_agg(with_counts):
    mesh = plsc.VectorSubcoreMesh(**_MESH)
    out_type = [
        jax.ShapeDtypeStruct((N_NODES, D), jnp.float32),
        jax.ShapeDtypeStruct((N_NODES, D), jnp.float32),
    ]
    if with_counts:
        out_type += [
            jax.ShapeDtypeStruct((N_NODES, CW), jnp.float32),
            jax.ShapeDtypeStruct((N_NODES, CW), jnp.float32),
        ]
    scratch = ([pltpu.VMEM((CHUNK,), jnp.int32)] * (2 * _G) +
               [pltpu.VMEM((CHUNK, D), jnp.float32)] * _G +
               [pltpu.VMEM((TAIL,), jnp.int32)] * 2 +
               [pltpu.VMEM((TAIL, D), jnp.float32)])
    if with_counts:
        scratch += [pltpu.VMEM((CHUNK, CW), jnp.float32)]
    scratch += [pltpu.VMEM_SHARED((N_NODES, D), jnp.float32)]
    if with_counts:
        scratch += [pltpu.VMEM_SHARED((N_NODES, CW), jnp.float32)]
    scratch += [pltpu.SemaphoreType.DMA] * 3
    return pl.kernel(
        functools.partial(_sc_agg_body, with_counts),
        out_type=tuple(out_type),
        mesh=mesh,
        scratch_types=tuple(scratch),
        compiler_params=_SC_PARAMS,
    )


_N2 = N_NODES // 2
_D2 = 2 * D
_BM = 1000
_GRID = _N2 // _BM


def _sc_cnt_body(dst_a, dst_b, zeros_cw, ones_hbm, *refs):
    cnt_a_out, cnt_b_out = refs[:2]
    refs = refs[2:]
    idx = refs[0:_G]
    idx_t, ones_v, cacc, sem_i, sem_s = refs[_G:]

    c = lax.axis_index("c")
    s = lax.axis_index("s")
    base = s * EPW

    def run_direction(dst_hbm, cnt_out):
        for t in range(NS):
            @pl.when(s == t)
            def _():
                sl = pl.ds(_ZOFF[t], _ZROWS[t])
                pltpu.sync_copy(zeros_cw.at[pl.ds(0, _ZROWS[t])],
                                cacc.at[sl])
        pltpu.sync_copy(ones_hbm, ones_v)
        plsc.subcore_barrier()

        def drain_scatters(ks):
            for k in ks:
                pltpu.make_async_copy(ones_v, cacc.at[idx[k]], sem_s).wait()

        def run_set(g, ks):
            @pl.when(g > 0)
            def _():
                drain_scatters(ks)
            dsi = []
            for k in ks:
                sl = pl.ds(base + (g * _G + k) * CHUNK, CHUNK)
                dsi.append(pltpu.async_copy(dst_hbm.at[sl], idx[k], sem_i))
            for j, k in enumerate(ks):
                dsi[j].wait()
                pltpu.async_copy(ones_v, cacc.at[idx[k]], sem_s, add=True)

        half = _G // 2
        set0 = list(range(half))
        set1 = list(range(half, _G))

        def group(g, _):
            run_set(g, set0)
            run_set(g, set1)
            return 0

        lax.fori_loop(0, _NGROUP, group, 0)
        drain_scatters(set0)
        drain_scatters(set1)

        sl = pl.ds(base + _NGROUP * _G * CHUNK, TAIL)
        pltpu.sync_copy(dst_hbm.at[sl], idx_t)
        pltpu.async_copy(ones_v.at[pl.ds(0, TAIL)], cacc.at[idx_t],
                         sem_s, add=True)
        pltpu.make_async_copy(ones_v.at[pl.ds(0, TAIL)], cacc.at[idx_t],
                              sem_s).wait()
        plsc.subcore_barrier()

        for t in range(NS):
            @pl.when(s == t)
            def _():
                sl = pl.ds(_ZOFF[t], _ZROWS[t])
                pltpu.sync_copy(cacc.at[sl], cnt_out.at[sl])

    @pl.when(c == 0)
    def _():
        run_direction(dst_a, cnt_a_out)

    @pl.when(c == 1)
    def _():
        run_direction(dst_b, cnt_b_out)


def _make_sc_cnt():
    mesh = plsc.VectorSubcoreMesh(**_MESH)
    out_type = (
        jax.ShapeDtypeStruct((N_NODES, CW), jnp.float32),
        jax.ShapeDtypeStruct((N_NODES, CW), jnp.float32),
    )
    scratch = ([pltpu.VMEM((CHUNK,), jnp.int32)] * _G +
               [pltpu.VMEM((TAIL,), jnp.int32)] +
               [pltpu.VMEM((CHUNK, CW), jnp.float32)] +
               [pltpu.VMEM_SHARED((N_NODES, CW), jnp.float32)] +
               [pltpu.SemaphoreType.DMA] * 2)
    return pl.kernel(
        _sc_cnt_body,
        out_type=out_type,
        mesh=mesh,
        scratch_types=tuple(scratch),
        compiler_params=_SC_PARAMS,
    )


def _proj_body(xc, wc, bc, xd, wd, bd, xs, ws, bs, oc, od, os_):
    dn2 = (((1,), (0,)), ((), ()))
    dnt = (((1,), (1,)), ((), ()))
    oc[...] = lax.dot_general(xc[...], wc[...], dn2,
                              preferred_element_type=jnp.float32) + bc[...]
    od[...] = lax.dot_general(xd[...], wd[...], dn2,
                              preferred_element_type=jnp.float32) + bd[...]
    os_[...] = jnp.maximum(
        lax.dot_general(xs[...], ws[...], dnt,
                        preferred_element_type=jnp.float32) + bs[...], 0.0)


def _conv_body(relu, agg_a, inv_a, xdst_a, wl_a, bl_a, wr_a,
               agg_b, inv_b, xdst_b, wl_b, bl_b, wr_b, oa, ob):
    dn2 = (((1,), (0,)), ((), ()))

    def one(agg, inv, xdst, wl, bl, wr, out):
        mean = agg[...] * inv[...]
        r = (lax.dot_general(mean, wl[...], dn2,
                             preferred_element_type=jnp.float32) + bl[...] +
             lax.dot_general(xdst[...], wr[...], dn2,
                             preferred_element_type=jnp.float32))
        out[...] = jnp.maximum(r, 0.0) if relu else r

    one(agg_a, inv_a, xdst_a, wl_a, bl_a, wr_a, oa)
    one(agg_b, inv_b, xdst_b, wl_b, bl_b, wr_b, ob)


def _proj_call(xc2, Pc, bc, xd2, Pd, bd, xs, Ws, bs):
    xspec = pl.BlockSpec((_BM, 2 * IN_DIM), lambda m: (m, 0))
    pspec = pl.BlockSpec((2 * IN_DIM, _D2), lambda m: (0, 0))
    b2spec = pl.BlockSpec((1, _D2), lambda m: (0, 0))
    o2spec = pl.BlockSpec((_BM, _D2), lambda m: (m, 0))
    sspec = pl.BlockSpec((2 * _BM, IN_DIM), lambda m: (m, 0))
    wsspec = pl.BlockSpec((D, IN_DIM), lambda m: (0, 0))
    bsspec = pl.BlockSpec((1, D), lambda m: (0, 0))
    osspec = pl.BlockSpec((2 * _BM, D), lambda m: (m, 0))
    return pl.pallas_call(
        _proj_body,
        grid=(_GRID,),
        in_specs=[xspec, pspec, b2spec, xspec, pspec, b2spec,
                  sspec, wsspec, bsspec],
        out_specs=[o2spec, o2spec, osspec],
        out_shape=[jax.ShapeDtypeStruct((_N2, _D2), jnp.float32)] * 2 +
                  [jax.ShapeDtypeStruct((N_NODES, D), jnp.float32)],
    )(xc2, Pc, bc, xd2, Pd, bd, xs, Ws, bs.reshape(1, D))


def _conv_call(relu, agg_a, inv_a, xdst_a, wl_a, bl_a, wr_a,
               agg_b, inv_b, xdst_b, wl_b, bl_b, wr_b):
    aspec = pl.BlockSpec((_BM, _D2), lambda m: (m, 0))
    wspec = pl.BlockSpec((_D2, _D2), lambda m: (0, 0))
    bspec = pl.BlockSpec((1, _D2), lambda m: (0, 0))
    oshape = jax.ShapeDtypeStruct((_N2, _D2), jnp.float32)
    return pl.pallas_call(
        functools.partial(_conv_body, relu),
        grid=(_GRID,),
        in_specs=[aspec, aspec, aspec, wspec, bspec, wspec] * 2,
        out_specs=[aspec] * 2,
        out_shape=[oshape] * 2,
    )(agg_a, inv_a, xdst_a, wl_a, bl_a, wr_a,
      agg_b, inv_b, xdst_b, wl_b, bl_b, wr_b)


def _blockdiag2(Wt):
    k, n = Wt.shape
    z = jnp.zeros((k, n), jnp.float32)
    return jnp.concatenate([
        jnp.concatenate([Wt, z], axis=1),
        jnp.concatenate([z, Wt], axis=1),
    ], axis=0)


def _inv128(cnt):
    inv = 1.0 / jnp.maximum(cnt[:, 0], 1.0)
    return jnp.repeat(inv.reshape(_N2, 2), D, axis=1)


def kernel(x_chemical, x_disease, x_side_effect, edge_index_treats,
           edge_index_rev_treats,
           Wp_c, bp_c, Wp_d, bp_d, Wp_s, bp_s,
           Wl1_td, bl1_td, Wr1_td, Wl1_dc, bl1_dc, Wr1_dc,
           Wl2_td, bl2_td, Wr2_td, Wl2_dc, bl2_dc, Wr2_dc):
    src_td = edge_index_treats[0]
    dst_td = edge_index_treats[1]
    src_dc = edge_index_rev_treats[0]
    dst_dc = edge_index_rev_treats[1]
    zeros64 = jnp.zeros((_ZROWS[0], D), jnp.float32)
    zeros_cw = jnp.zeros((_ZROWS[0], CW), jnp.float32)
    ones = jnp.ones((CHUNK, CW), jnp.float32)

    def bd2(b):
        return jnp.concatenate([b, b]).reshape(1, _D2)

    cnt_td, cnt_dc = _make_sc_cnt()(dst_td, dst_dc, zeros_cw, ones)
    inv_td = _inv128(cnt_td)
    inv_dc = _inv128(cnt_dc)

    xc2, xd2, s1 = _proj_call(
        x_chemical.reshape(_N2, 2 * IN_DIM), _blockdiag2(Wp_c.T), bd2(bp_c),
        x_disease.reshape(_N2, 2 * IN_DIM), _blockdiag2(Wp_d.T), bd2(bp_d),
        x_side_effect, Wp_s, bp_s)

    sc_agg = _make_sc_agg(False)
    agg_td, agg_dc = sc_agg(
        src_td, dst_td, xc2.reshape(N_NODES, D),
        src_dc, dst_dc, xd2.reshape(N_NODES, D),
        zeros64, zeros_cw, ones)

    d1, c1 = _conv_call(
        True,
        agg_td.reshape(_N2, _D2), inv_td, xd2,
        _blockdiag2(Wl1_td.T), bd2(bl1_td), _blockdiag2(Wr1_td.T),
        agg_dc.reshape(_N2, _D2), inv_dc, xc2,
        _blockdiag2(Wl1_dc.T), bd2(bl1_dc), _blockdiag2(Wr1_dc.T))

    agg2_td, agg2_dc = sc_agg(
        src_td, dst_td, c1.reshape(N_NODES, D),
        src_dc, dst_dc, d1.reshape(N_NODES, D),
        zeros64, zeros_cw, ones)

    d2, c2 = _conv_call(
        False,
        agg2_td.reshape(_N2, _D2), inv_td, d1,
        _blockdiag2(Wl2_td.T), bd2(bl2_td), _blockdiag2(Wr2_td.T),
        agg2_dc.reshape(_N2, _D2), inv_dc, c1,
        _blockdiag2(Wl2_dc.T), bd2(bl2_dc), _blockdiag2(Wr2_dc.T))

    return c2.reshape(N_NODES, D), d2.reshape(N_NODES, D), s1


# speedup vs baseline: 1.0115x; 1.0115x over previous
import functools

import jax
import jax.numpy as jnp
from jax import lax
from jax.experimental import pallas as pl
from jax.experimental.pallas import tpu as pltpu
from jax.experimental.pallas import tpu_sc as plsc

N_NODES = 10000
IN_DIM = 128
D = 64
CW = 8
E = 640000

NC = 2
NS = 16
EPW = E // NS
CHUNK = 128
_G = 8
_NGROUP = 39
TAIL = EPW - _NGROUP * _G * CHUNK

_ZROWS = [640] * 15 + [400]
_ZOFF = [640 * i for i in range(16)]

_SC_PARAMS = pltpu.CompilerParams(use_tc_tiling_on_sc=False)
_MESH = dict(core_axis_name="c", subcore_axis_name="s",
             num_cores=NC, num_subcores=NS)


def _sc_agg_body(with_counts, src_a, dst_a, tab_a, src_b, dst_b, tab_b,
                 zeros64, zeros_cw, ones_hbm, *refs):
    if with_counts:
        agg_a_out, agg_b_out, cnt_a_out, cnt_b_out = refs[:4]
        refs = refs[4:]
    else:
        agg_a_out, agg_b_out = refs[:2]
        cnt_a_out = cnt_b_out = None
        refs = refs[2:]
    isx = refs[0:_G]
    idx = refs[_G:2 * _G]
    rws = refs[2 * _G:3 * _G]
    isx_t, idx_t, rws_t = refs[3 * _G:3 * _G + 3]
    refs = refs[3 * _G + 3:]
    if with_counts:
        ones_v, acc, cacc, sem_i, sem_g, sem_s = refs
    else:
        acc, sem_i, sem_g, sem_s = refs
        cacc = None

    c = lax.axis_index("c")
    s = lax.axis_index("s")
    base = s * EPW

    def run_direction(src_hbm, dst_hbm, tab_hbm, agg_out, cnt_out):
        for t in range(NS):
            @pl.when(s == t)
            def _():
                sl = pl.ds(_ZOFF[t], _ZROWS[t])
                pltpu.sync_copy(zeros64.at[sl], acc.at[sl])
                if with_counts:
                    pltpu.sync_copy(zeros_cw.at[sl], cacc.at[sl])
        if with_counts:
            pltpu.sync_copy(ones_hbm, ones_v)
        plsc.subcore_barrier()

        def drain_scatters(ks):
            for k in ks:
                pltpu.make_async_copy(rws[k], acc.at[idx[k]], sem_s).wait()
                if with_counts:
                    pltpu.make_async_copy(ones_v, cacc.at[idx[k]],
                                          sem_s).wait()

        def run_set(g, ks):
            @pl.when(g > 0)
            def _():
                drain_scatters(ks)
            dsi = []
            for k in ks:
                sl = pl.ds(base + (g * _G + k) * CHUNK, CHUNK)
                dsi.append(pltpu.async_copy(src_hbm.at[sl], isx[k], sem_i))
                dsi.append(pltpu.async_copy(dst_hbm.at[sl], idx[k], sem_i))
            dsg = []
            for j, k in enumerate(ks):
                dsi[2 * j].wait()
                dsi[2 * j + 1].wait()
                dsg.append(pltpu.async_copy(tab_hbm.at[isx[k]], rws[k],
                                            sem_g))
            for j, k in enumerate(ks):
                dsg[j].wait()
                pltpu.async_copy(rws[k], acc.at[idx[k]], sem_s, add=True)
                if with_counts:
                    pltpu.async_copy(ones_v, cacc.at[idx[k]], sem_s,
                                     add=True)

        half = _G // 2
        set0 = list(range(half))
        set1 = list(range(half, _G))

        def group(g, _):
            run_set(g, set0)
            run_set(g, set1)
            return 0

        lax.fori_loop(0, _NGROUP, group, 0)
        drain_scatters(set0)
        drain_scatters(set1)

        sl = pl.ds(base + _NGROUP * _G * CHUNK, TAIL)
        pltpu.sync_copy(src_hbm.at[sl], isx_t)
        pltpu.sync_copy(dst_hbm.at[sl], idx_t)
        pltpu.async_copy(tab_hbm.at[isx_t], rws_t, sem_g).wait()
        pltpu.async_copy(rws_t, acc.at[idx_t], sem_s, add=True)
        if with_counts:
            pltpu.async_copy(ones_v.at[pl.ds(0, TAIL)], cacc.at[idx_t],
                             sem_s, add=True)
            pltpu.make_async_copy(ones_v.at[pl.ds(0, TAIL)],
                                  cacc.at[idx_t], sem_s).wait()
        pltpu.make_async_copy(rws_t, acc.at[idx_t], sem_s).wait()
        plsc.subcore_barrier()

        for t in range(NS):
            @pl.when(s == t)
            def _():
                sl = pl.ds(_ZOFF[t], _ZROWS[t])
                pltpu.sync_copy(acc.at[sl], agg_out.at[sl])
                if with_counts:
                    pltpu.sync_copy(cacc.at[sl], cnt_out.at[sl])

    @pl.when(c == 0)
    def _():
        run_direction(src_a, dst_a, tab_a, agg_a_out, cnt_a_out)

    @pl.when(c == 1)
    def _():
        run_direction(src_b, dst_b, tab_b, agg_b_out, cnt_b_out)


def _make_sc_agg(with_counts):
    mesh = plsc.VectorSubcoreMesh(**_MESH)
    out_type = [
        jax.ShapeDtypeStruct((N_NODES, D), jnp.float32),
        jax.ShapeDtypeStruct((N_NODES, D), jnp.float32),
    ]
    if with_counts:
        out_type += [
            jax.ShapeDtypeStruct((N_NODES, CW), jnp.float32),
            jax.ShapeDtypeStruct((N_NODES, CW), jnp.float32),
        ]
    scratch = ([pltpu.VMEM((CHUNK,), jnp.int32)] * (2 * _G) +
               [pltpu.VMEM((CHUNK, D), jnp.float32)] * _G +
               [pltpu.VMEM((TAIL,), jnp.int32)] * 2 +
               [pltpu.VMEM((TAIL, D), jnp.float32)])
    if with_counts:
        scratch += [pltpu.VMEM((CHUNK, CW), jnp.float32)]
    scratch += [pltpu.VMEM_SHARED((N_NODES, D), jnp.float32)]
    if with_counts:
        scratch += [pltpu.VMEM_SHARED((N_NODES, CW), jnp.float32)]
    scratch += [pltpu.SemaphoreType.DMA] * 3
    return pl.kernel(
        functools.partial(_sc_agg_body, with_counts),
        out_type=tuple(out_type),
        mesh=mesh,
        scratch_types=tuple(scratch),
        compiler_params=_SC_PARAMS,
    )


_N2 = N_NODES // 2
_D2 = 2 * D
_BM = 1000
_GRID = _N2 // _BM


def _sc_cnt_body(dst_a, dst_b, zeros_cw, ones_hbm, *refs):
    cnt_a_out, cnt_b_out = refs[:2]
    refs = refs[2:]
    idx = refs[0:_G]
    idx_t, ones_v, cacc, sem_i, sem_s = refs[_G:]

    c = lax.axis_index("c")
    s = lax.axis_index("s")
    base = s * EPW

    def run_direction(dst_hbm, cnt_out):
        for t in range(NS):
            @pl.when(s == t)
            def _():
                sl = pl.ds(_ZOFF[t], _ZROWS[t])
                pltpu.sync_copy(zeros_cw.at[sl], cacc.at[sl])
        pltpu.sync_copy(ones_hbm, ones_v)
        plsc.subcore_barrier()

        def drain_scatters(ks):
            for k in ks:
                pltpu.make_async_copy(ones_v, cacc.at[idx[k]], sem_s).wait()

        def run_set(g, ks):
            @pl.when(g > 0)
            def _():
                drain_scatters(ks)
            dsi = []
            for k in ks:
                sl = pl.ds(base + (g * _G + k) * CHUNK, CHUNK)
                dsi.append(pltpu.async_copy(dst_hbm.at[sl], idx[k], sem_i))
            for j, k in enumerate(ks):
                dsi[j].wait()
                pltpu.async_copy(ones_v, cacc.at[idx[k]], sem_s, add=True)

        half = _G // 2
        set0 = list(range(half))
        set1 = list(range(half, _G))

        def group(g, _):
            run_set(g, set0)
            run_set(g, set1)
            return 0

        lax.fori_loop(0, _NGROUP, group, 0)
        drain_scatters(set0)
        drain_scatters(set1)

        sl = pl.ds(base + _NGROUP * _G * CHUNK, TAIL)
        pltpu.sync_copy(dst_hbm.at[sl], idx_t)
        pltpu.async_copy(ones_v.at[pl.ds(0, TAIL)], cacc.at[idx_t],
                         sem_s, add=True)
        pltpu.make_async_copy(ones_v.at[pl.ds(0, TAIL)], cacc.at[idx_t],
                              sem_s).wait()
        plsc.subcore_barrier()

        for t in range(NS):
            @pl.when(s == t)
            def _():
                sl = pl.ds(_ZOFF[t], _ZROWS[t])
                pltpu.sync_copy(cacc.at[sl], cnt_out.at[sl])

    @pl.when(c == 0)
    def _():
        run_direction(dst_a, cnt_a_out)

    @pl.when(c == 1)
    def _():
        run_direction(dst_b, cnt_b_out)


def _make_sc_cnt():
    mesh = plsc.VectorSubcoreMesh(**_MESH)
    out_type = (
        jax.ShapeDtypeStruct((N_NODES, CW), jnp.float32),
        jax.ShapeDtypeStruct((N_NODES, CW), jnp.float32),
    )
    scratch = ([pltpu.VMEM((CHUNK,), jnp.int32)] * _G +
               [pltpu.VMEM((TAIL,), jnp.int32)] +
               [pltpu.VMEM((CHUNK, CW), jnp.float32)] +
               [pltpu.VMEM_SHARED((N_NODES, CW), jnp.float32)] +
               [pltpu.SemaphoreType.DMA] * 2)
    return pl.kernel(
        _sc_cnt_body,
        out_type=out_type,
        mesh=mesh,
        scratch_types=tuple(scratch),
        compiler_params=_SC_PARAMS,
    )


def _proj_body(xc, wc, bc, xd, wd, bd, xs, ws, bs, oc, od, os_):
    dn2 = (((1,), (0,)), ((), ()))
    dnt = (((1,), (1,)), ((), ()))
    oc[...] = lax.dot_general(xc[...], wc[...], dn2,
                              preferred_element_type=jnp.float32) + bc[...]
    od[...] = lax.dot_general(xd[...], wd[...], dn2,
                              preferred_element_type=jnp.float32) + bd[...]
    os_[...] = jnp.maximum(
        lax.dot_general(xs[...], ws[...], dnt,
                        preferred_element_type=jnp.float32) + bs[...], 0.0)


def _conv_body(relu, agg_a, inv_a, xdst_a, wl_a, bl_a, wr_a,
               agg_b, inv_b, xdst_b, wl_b, bl_b, wr_b, oa, ob):
    dn2 = (((1,), (0,)), ((), ()))

    def one(agg, inv, xdst, wl, bl, wr, out):
        mean = agg[...] * inv[...]
        r = (lax.dot_general(mean, wl[...], dn2,
                             preferred_element_type=jnp.float32) + bl[...] +
             lax.dot_general(xdst[...], wr[...], dn2,
                             preferred_element_type=jnp.float32))
        out[...] = jnp.maximum(r, 0.0) if relu else r

    one(agg_a, inv_a, xdst_a, wl_a, bl_a, wr_a, oa)
    one(agg_b, inv_b, xdst_b, wl_b, bl_b, wr_b, ob)


def _proj_call(xc2, Pc, bc, xd2, Pd, bd, xs, Ws, bs):
    xspec = pl.BlockSpec((_BM, 2 * IN_DIM), lambda m: (m, 0))
    pspec = pl.BlockSpec((2 * IN_DIM, _D2), lambda m: (0, 0))
    b2spec = pl.BlockSpec((1, _D2), lambda m: (0, 0))
    o2spec = pl.BlockSpec((_BM, _D2), lambda m: (m, 0))
    sspec = pl.BlockSpec((2 * _BM, IN_DIM), lambda m: (m, 0))
    wsspec = pl.BlockSpec((D, IN_DIM), lambda m: (0, 0))
    bsspec = pl.BlockSpec((1, D), lambda m: (0, 0))
    osspec = pl.BlockSpec((2 * _BM, D), lambda m: (m, 0))
    return pl.pallas_call(
        _proj_body,
        grid=(_GRID,),
        in_specs=[xspec, pspec, b2spec, xspec, pspec, b2spec,
                  sspec, wsspec, bsspec],
        out_specs=[o2spec, o2spec, osspec],
        out_shape=[jax.ShapeDtypeStruct((_N2, _D2), jnp.float32)] * 2 +
                  [jax.ShapeDtypeStruct((N_NODES, D), jnp.float32)],
    )(xc2, Pc, bc, xd2, Pd, bd, xs, Ws, bs.reshape(1, D))


def _conv_call(relu, agg_a, inv_a, xdst_a, wl_a, bl_a, wr_a,
               agg_b, inv_b, xdst_b, wl_b, bl_b, wr_b):
    aspec = pl.BlockSpec((_BM, _D2), lambda m: (m, 0))
    wspec = pl.BlockSpec((_D2, _D2), lambda m: (0, 0))
    bspec = pl.BlockSpec((1, _D2), lambda m: (0, 0))
    oshape = jax.ShapeDtypeStruct((_N2, _D2), jnp.float32)
    return pl.pallas_call(
        functools.partial(_conv_body, relu),
        grid=(_GRID,),
        in_specs=[aspec, aspec, aspec, wspec, bspec, wspec] * 2,
        out_specs=[aspec] * 2,
        out_shape=[oshape] * 2,
    )(agg_a, inv_a, xdst_a, wl_a, bl_a, wr_a,
      agg_b, inv_b, xdst_b, wl_b, bl_b, wr_b)


def _blockdiag2(Wt):
    k, n = Wt.shape
    z = jnp.zeros((k, n), jnp.float32)
    return jnp.concatenate([
        jnp.concatenate([Wt, z], axis=1),
        jnp.concatenate([z, Wt], axis=1),
    ], axis=0)


def _inv128(cnt):
    inv = 1.0 / jnp.maximum(cnt[:, 0], 1.0)
    return jnp.repeat(inv.reshape(_N2, 2), D, axis=1)


def kernel(x_chemical, x_disease, x_side_effect, edge_index_treats,
           edge_index_rev_treats,
           Wp_c, bp_c, Wp_d, bp_d, Wp_s, bp_s,
           Wl1_td, bl1_td, Wr1_td, Wl1_dc, bl1_dc, Wr1_dc,
           Wl2_td, bl2_td, Wr2_td, Wl2_dc, bl2_dc, Wr2_dc):
    src_td = edge_index_treats[0]
    dst_td = edge_index_treats[1]
    src_dc = edge_index_rev_treats[0]
    dst_dc = edge_index_rev_treats[1]
    zeros64 = jnp.zeros((N_NODES, D), jnp.float32)
    zeros_cw = jnp.zeros((N_NODES, CW), jnp.float32)
    ones = jnp.ones((CHUNK, CW), jnp.float32)

    def bd2(b):
        return jnp.concatenate([b, b]).reshape(1, _D2)

    cnt_td, cnt_dc = _make_sc_cnt()(dst_td, dst_dc, zeros_cw, ones)
    inv_td = _inv128(cnt_td)
    inv_dc = _inv128(cnt_dc)

    xc2, xd2, s1 = _proj_call(
        x_chemical.reshape(_N2, 2 * IN_DIM), _blockdiag2(Wp_c.T), bd2(bp_c),
        x_disease.reshape(_N2, 2 * IN_DIM), _blockdiag2(Wp_d.T), bd2(bp_d),
        x_side_effect, Wp_s, bp_s)

    sc_agg = _make_sc_agg(False)
    agg_td, agg_dc = sc_agg(
        src_td, dst_td, xc2.reshape(N_NODES, D),
        src_dc, dst_dc, xd2.reshape(N_NODES, D),
        zeros64, zeros_cw, ones)

    d1, c1 = _conv_call(
        True,
        agg_td.reshape(_N2, _D2), inv_td, xd2,
        _blockdiag2(Wl1_td.T), bd2(bl1_td), _blockdiag2(Wr1_td.T),
        agg_dc.reshape(_N2, _D2), inv_dc, xc2,
        _blockdiag2(Wl1_dc.T), bd2(bl1_dc), _blockdiag2(Wr1_dc.T))

    agg2_td, agg2_dc = sc_agg(
        src_td, dst_td, c1.reshape(N_NODES, D),
        src_dc, dst_dc, d1.reshape(N_NODES, D),
        zeros64, zeros_cw, ones)

    d2, c2 = _conv_call(
        False,
        agg2_td.reshape(_N2, _D2), inv_td, d1,
        _blockdiag2(Wl2_td.T), bd2(bl2_td), _blockdiag2(Wr2_td.T),
        agg2_dc.reshape(_N2, _D2), inv_dc, c1,
        _blockdiag2(Wl2_dc.T), bd2(bl2_dc), _blockdiag2(Wr2_dc.T))

    return c2.reshape(N_NODES, D), d2.reshape(N_NODES, D), s1
